# Initial kernel scaffold; baseline (speedup 1.0000x reference)
import jax
import jax.numpy as jnp
from jax.experimental import pallas as pl


def kernel(phi, idx_durations, events):
    raise NotImplementedError("write your pallas kernel here")


# speedup vs baseline: 5.1908x; 5.1908x over previous
import functools

import jax
import jax.numpy as jnp
from jax.experimental import pallas as pl
from jax.experimental.pallas import tpu as pltpu


def _tc_body(phi_ref, d_ref, e_ref, out_ref, *, Q, K):
    p = phi_ref[...]
    NB = p.shape[0]
    zs = [p[:, q * K:(q + 1) * K] for q in range(Q)]
    s = zs[0]
    for q in range(1, Q):
        s = s + zs[q]
    zlast = 1.0 - s
    m = zlast
    for z in zs:
        m = jnp.maximum(m, z)
    se = jnp.exp(zlast - m)
    for z in zs:
        se = se + jnp.exp(z - m)
    lse = m + jnp.log(se)

    d = d_ref[0, 0, :].reshape(NB, 1)
    e = e_ref[0, 0, :].reshape(NB, 1)
    kio = jax.lax.broadcasted_iota(jnp.int32, (NB, K), 1)
    c = jnp.where(kio <= d, lse - zlast, 0.0)

    ze = zs[Q - 1]
    for q in range(Q - 2, -1, -1):
        ze = jnp.where(e == q + 1, zs[q], ze)
    corr = jnp.where((kio == d) & (e != 0), zlast - ze, 0.0)

    total = jnp.sum(c) + jnp.sum(corr)

    @pl.when(pl.program_id(0) == 0)
    def _init():
        out_ref[0, 0] = 0.0

    out_ref[0, 0] += total


def kernel(phi, idx_durations, events):
    N, Q, K = phi.shape
    NB = 256
    nblk = N // NB
    phi2 = phi.reshape(N, Q * K)
    d3 = idx_durations.astype(jnp.int32).reshape(nblk, 1, NB)
    e3 = events.astype(jnp.int32).reshape(nblk, 1, NB)
    out = pl.pallas_call(
        functools.partial(_tc_body, Q=Q, K=K),
        grid=(nblk,),
        in_specs=[
            pl.BlockSpec((NB, Q * K), lambda i: (i, 0)),
            pl.BlockSpec((1, 1, NB), lambda i: (i, 0, 0)),
            pl.BlockSpec((1, 1, NB), lambda i: (i, 0, 0)),
        ],
        out_specs=pl.BlockSpec(memory_space=pltpu.SMEM),
        out_shape=jax.ShapeDtypeStruct((1, 1), jnp.float32),
        compiler_params=pltpu.CompilerParams(
            dimension_semantics=("arbitrary",),
        ),
    )(phi2, d3, e3)
    return out[0, 0] / N


# speedup vs baseline: 5.7193x; 1.1018x over previous
import functools

import jax
import jax.numpy as jnp
from jax.experimental import pallas as pl
from jax.experimental.pallas import tpu as pltpu


def _tc_body(phi_ref, d_ref, e_ref, out_ref, *, Q, K):
    p = phi_ref[...]
    NB = p.shape[0]
    zs = [p[:, q * K:(q + 1) * K] for q in range(Q)]
    s = zs[0]
    for q in range(1, Q):
        s = s + zs[q]
    zlast = 1.0 - s
    m = zlast
    for z in zs:
        m = jnp.maximum(m, z)
    se = jnp.exp(zlast - m)
    for z in zs:
        se = se + jnp.exp(z - m)
    lse = m + jnp.log(se)

    d = d_ref[0, 0, :].reshape(NB, 1)
    e = e_ref[0, 0, :].reshape(NB, 1)
    kio = jax.lax.broadcasted_iota(jnp.int32, (NB, K), 1)
    c = jnp.where(kio <= d, lse - zlast, 0.0)

    ze = zs[Q - 1]
    for q in range(Q - 2, -1, -1):
        ze = jnp.where(e == q + 1, zs[q], ze)
    corr = jnp.where((kio == d) & (e != 0), zlast - ze, 0.0)

    total = jnp.sum(c) + jnp.sum(corr)

    @pl.when(pl.program_id(0) == 0)
    def _init():
        out_ref[0, 0] = 0.0

    out_ref[0, 0] += total


def kernel(phi, idx_durations, events):
    N, Q, K = phi.shape
    NB = 512
    nblk = N // NB
    phi2 = phi.reshape(N, Q * K)
    d3 = idx_durations.astype(jnp.int32).reshape(nblk, 1, NB)
    e3 = events.astype(jnp.int32).reshape(nblk, 1, NB)
    out = pl.pallas_call(
        functools.partial(_tc_body, Q=Q, K=K),
        grid=(nblk,),
        in_specs=[
            pl.BlockSpec((NB, Q * K), lambda i: (i, 0)),
            pl.BlockSpec((1, 1, NB), lambda i: (i, 0, 0)),
            pl.BlockSpec((1, 1, NB), lambda i: (i, 0, 0)),
        ],
        out_specs=pl.BlockSpec(memory_space=pltpu.SMEM),
        out_shape=jax.ShapeDtypeStruct((1, 1), jnp.float32),
        compiler_params=pltpu.CompilerParams(
            dimension_semantics=("arbitrary",),
        ),
    )(phi2, d3, e3)
    return out[0, 0] / N


# speedup vs baseline: 6.0053x; 1.0500x over previous
import functools

import jax
import jax.numpy as jnp
from jax.experimental import pallas as pl
from jax.experimental.pallas import tpu as pltpu

_S = 2


def _partial_sum(p, d, e, *, Q, K):
    NB = p.shape[0]
    zs = [p[:, q * K:(q + 1) * K] for q in range(Q)]
    s = zs[0]
    for q in range(1, Q):
        s = s + zs[q]
    zlast = 1.0 - s
    m = zlast
    for z in zs:
        m = jnp.maximum(m, z)
    se = jnp.exp(zlast - m)
    for z in zs:
        se = se + jnp.exp(z - m)
    lse = m + jnp.log(se)

    d = d.reshape(NB, 1)
    e = e.reshape(NB, 1)
    kio = jax.lax.broadcasted_iota(jnp.int32, (NB, K), 1)
    c = jnp.where(kio <= d, lse - zlast, 0.0)

    ze = zs[Q - 1]
    for q in range(Q - 2, -1, -1):
        ze = jnp.where(e == q + 1, zs[q], ze)
    corr = jnp.where((kio == d) & (e != 0), zlast - ze, 0.0)
    return jnp.sum(c) + jnp.sum(corr)


def _tc_body(*refs, Q, K):
    phi_refs = refs[:_S]
    d_refs = refs[_S:2 * _S]
    e_refs = refs[2 * _S:3 * _S]
    out_ref = refs[3 * _S]
    total = 0.0
    for s in range(_S):
        total += _partial_sum(phi_refs[s][...], d_refs[s][0, 0, :],
                              e_refs[s][0, 0, :], Q=Q, K=K)

    @pl.when(pl.program_id(0) == 0)
    def _init():
        out_ref[0, 0] = 0.0

    out_ref[0, 0] += total


def kernel(phi, idx_durations, events):
    N, Q, K = phi.shape
    NB = 512
    nblk = N // NB
    g = nblk // _S
    phi2 = phi.reshape(N, Q * K)
    d3 = idx_durations.astype(jnp.int32).reshape(nblk, 1, NB)
    e3 = events.astype(jnp.int32).reshape(nblk, 1, NB)

    def phi_map(s):
        return lambda i: (i + s * g, 0)

    def de_map(s):
        return lambda i: (i + s * g, 0, 0)

    out = pl.pallas_call(
        functools.partial(_tc_body, Q=Q, K=K),
        grid=(g,),
        in_specs=(
            [pl.BlockSpec((NB, Q * K), phi_map(s)) for s in range(_S)]
            + [pl.BlockSpec((1, 1, NB), de_map(s)) for s in range(_S)]
            + [pl.BlockSpec((1, 1, NB), de_map(s)) for s in range(_S)]
        ),
        out_specs=pl.BlockSpec(memory_space=pltpu.SMEM),
        out_shape=jax.ShapeDtypeStruct((1, 1), jnp.float32),
        compiler_params=pltpu.CompilerParams(
            dimension_semantics=("arbitrary",),
        ),
    )(*([phi2] * _S + [d3] * _S + [e3] * _S))
    return out[0, 0] / N
